# baseline (speedup 1.0000x reference)
import jax
import jax.numpy as jnp
from jax.experimental import pallas as pl

_BLK = 4096


def _blk(x_ref, o_ref):
    xb = x_ref[...]
    n = xb.shape[0]
    i128 = jax.lax.broadcasted_iota(jnp.int32, (n, 128), 1)
    m32 = i128 & 31
    ia = ((m32 >> 2) & 3) + ((m32 & 16) >> 1)
    ib = 4 + (m32 & 3) + ((m32 & 16) >> 1)
    p128 = jnp.minimum(jnp.take_along_axis(xb, ia, axis=1),
                       jnp.take_along_axis(xb, ib, axis=1))
    lo = 16 + (i128 & 15)
    pl_lo = jnp.take_along_axis(p128, lo, axis=1)
    hi0 = i128 >> 4
    hi1 = 8 + hi0
    o_ref[:, 0:128] = jnp.minimum(jnp.take_along_axis(p128, hi0, axis=1), pl_lo)
    o_ref[:, 128:256] = jnp.minimum(jnp.take_along_axis(p128, hi1, axis=1), pl_lo)


def kernel(x, indexes):
    b, n_in, n_mf = x.shape
    r = indexes.shape[0]
    del indexes
    xf = x.reshape(b, n_in * n_mf)
    return pl.pallas_call(
        _blk,
        grid=(b // _BLK,),
        in_specs=[pl.BlockSpec((_BLK, n_in * n_mf), lambda i: (i, 0))],
        out_specs=pl.BlockSpec((_BLK, r), lambda i: (i, 0)),
        out_shape=jax.ShapeDtypeStruct((b, r), jnp.float32),
    )(xf)
